# baseline (speedup 1.0000x reference)
import functools

import jax
import jax.numpy as jnp
from jax import lax
from jax.experimental import pallas as pl
from jax.experimental.pallas import tpu as pltpu
from jax.experimental.pallas import tpu_sc as plsc

N = 100000
NR = 100352
STRIPE = NR // 16
E = 6400000
ET = 401280
E_PAD = ET * 16
DEG_CH = 3344
DEG_ET = E_PAD // 32
DEG_NITER = DEG_ET // DEG_CH
EPS = 1e-5

_MESH = plsc.VectorSubcoreMesh(core_axis_name="c", subcore_axis_name="s")


def _deg_body(dst_hbm, zeros1, hist_out, hist, db0, db1, sem0, sem1):
    c = lax.axis_index("c")
    s = lax.axis_index("s")
    wid = c * 16 + s
    dbufs = (db0, db1)
    sems = (sem0, sem1)
    pltpu.sync_copy(zeros1, hist)
    ones = jnp.full((16,), 1.0, jnp.float32)

    def fire(t, b):
        pltpu.async_copy(
            dst_hbm.at[pl.ds(wid * DEG_ET + t * DEG_CH, DEG_CH)],
            dbufs[b], sems[b])

    def drain(b):
        pltpu.make_async_copy(
            dst_hbm.at[pl.ds(0, DEG_CH)], dbufs[b], sems[b]).wait()

    fire(0, 0)

    @pl.loop(0, DEG_NITER // 2)
    def _(u):
        for r in range(2):
            t = 2 * u + r

            @pl.when(t + 1 < DEG_NITER)
            def _():
                fire(t + 1, 1 - r)

            drain(r)
            for j in range(DEG_CH // 16):
                idx = dbufs[r][pl.ds(j * 16, 16)]
                plsc.addupdate_scatter(hist, [idx], ones)

    pltpu.sync_copy(hist, hist_out.at[pl.ds(wid * NR, NR)])


_deg_call = pl.kernel(
    _deg_body,
    out_type=jax.ShapeDtypeStruct((32 * NR,), jnp.float32),
    mesh=_MESH,
    compiler_params=pltpu.CompilerParams(needs_layout_passes=False),
    scratch_types=[
        pltpu.VMEM((NR,), jnp.float32),
        pltpu.VMEM((DEG_CH,), jnp.int32),
        pltpu.VMEM((DEG_CH,), jnp.int32),
        pltpu.SemaphoreType.DMA,
        pltpu.SemaphoreType.DMA,
    ],
)


GRP = 384
GK = GRP // 128
NGRP = ET // GRP
IB = 4
MAIN = (NGRP - 8) // IB


def _make_spmm(P):
    PH = P // 2

    def body(src_hbm, dst2_hbm, zeros2, *rest):
        zrefs = rest[:P]
        srefs = rest[P:2 * P]
        acc = rest[2 * P]
        pos = 2 * P + 1
        src_ch = rest[pos:pos + IB]
        pos += IB
        dst_ch = rest[pos:pos + IB]
        pos += IB
        row_ch = [rest[pos + b * GK:pos + (b + 1) * GK] for b in range(IB)]
        pos += IB * GK
        i_sem = rest[pos:pos + IB]
        g_sem = rest[pos + IB:pos + 2 * IB]
        s_sem = rest[pos + 2 * IB:pos + 3 * IB]
        c = lax.axis_index("c")
        s = lax.axis_index("s")

        def fire_i(u, b):
            base = s * ET + u * GRP
            pltpu.async_copy(src_hbm.at[pl.ds(base, GRP)], src_ch[b], i_sem[b])
            pltpu.async_copy(dst2_hbm.at[pl.ds(base // 128, GK)], dst_ch[b],
                             i_sem[b])

        def wait_i(b):
            pltpu.make_async_copy(
                src_hbm.at[pl.ds(0, GRP)], src_ch[b], i_sem[b]).wait()
            pltpu.make_async_copy(
                dst2_hbm.at[pl.ds(0, GK)], dst_ch[b], i_sem[b]).wait()

        def fire_g(b, Z):
            for m in range(GK):
                pltpu.async_copy(
                    Z.at[src_ch[b].at[pl.ds(128 * m, 128)]], row_ch[b][m],
                    g_sem[b])

        def wait_g(b, Z):
            for m in range(GK):
                pltpu.make_async_copy(
                    Z.at[src_ch[b].at[pl.ds(128 * m, 128)]], row_ch[b][m],
                    g_sem[b]).wait()

        def fire_s(b):
            for m in range(GK):
                pltpu.async_copy(row_ch[b][m], acc.at[dst_ch[b].at[m]],
                                 s_sem[b], add=True)

        def wait_s(b):
            for m in range(GK):
                pltpu.make_async_copy(row_ch[b][m], acc.at[dst_ch[b].at[m]],
                                      s_sem[b]).wait()

        for cv in range(2):
            @pl.when(c == cv)
            def _():
                for k in range(PH):
                    q = cv * PH + k
                    Z, S = zrefs[q], srefs[q]
                    pltpu.sync_copy(zeros2, acc.at[pl.ds(s * STRIPE, STRIPE)])
                    plsc.subcore_barrier()

                    def group(u, ib, has_gm1, has_sm2, has_ip2):
                        wait_i(ib)
                        fire_g(ib, Z)
                        if has_gm1:
                            wait_g((ib - 1) % IB, Z)
                            fire_s((ib - 1) % IB)
                        if has_sm2:
                            wait_s((ib - 2) % IB)
                        if has_ip2:
                            fire_i(u + 2, (ib + 2) % IB)

                    fire_i(0, 0)
                    fire_i(1, 1)
                    group(0, 0, False, False, True)
                    group(1, 1, True, False, True)

                    @pl.loop(0, MAIN)
                    def _(v):
                        for r in range(IB):
                            group(2 + IB * v + r, (2 + r) % IB,
                                  True, True, True)

                    for u in range(2 + IB * MAIN, NGRP):
                        group(u, u % IB, True, True, u + 2 < NGRP)
                    wait_g((NGRP - 1) % IB, Z)
                    fire_s((NGRP - 1) % IB)
                    wait_s((NGRP - 2) % IB)
                    wait_s((NGRP - 1) % IB)
                    plsc.subcore_barrier()
                    pltpu.sync_copy(acc.at[pl.ds(s * STRIPE, STRIPE)],
                                    S.at[pl.ds(s * STRIPE, STRIPE)])
                    plsc.subcore_barrier()

    return pl.kernel(
        body,
        out_type=[jax.ShapeDtypeStruct((NR, 16), jnp.float32)] * P,
        mesh=_MESH,
        compiler_params=pltpu.CompilerParams(
            needs_layout_passes=False, use_tc_tiling_on_sc=False),
        scratch_types=(
            [pltpu.VMEM_SHARED((NR, 16), jnp.float32)]
            + [pltpu.VMEM((GRP,), jnp.int32) for _ in range(IB)]
            + [pltpu.VMEM((GK, 128), jnp.int32) for _ in range(IB)]
            + [pltpu.VMEM((128, 16), jnp.float32) for _ in range(IB * GK)]
            + [pltpu.SemaphoreType.DMA for _ in range(3 * IB)]
        ),
    )


_spmm2 = _make_spmm(2)
_spmm4 = _make_spmm(4)


def _disk(hist_ref, dis_ref):
    deg = jnp.sum(hist_ref[...], axis=0) + 1.0
    dis_ref[...] = lax.rsqrt(deg)


def _tc_dis(hist32):
    h = hist32.reshape(32, NR // 128, 128)
    dis = pl.pallas_call(
        _disk,
        out_shape=jax.ShapeDtypeStruct((NR // 128, 128), jnp.float32),
    )(h)
    return dis.reshape(NR, 1)


_BN_ = 3584
_NB_ = NR // _BN_


def _z1k(x_ref, w_ref, dis_ref, o0, o1):
    z = jnp.dot(x_ref[...], w_ref[...], preferred_element_type=jnp.float32)
    z = z * dis_ref[...]
    o0[...] = z[:, :16]
    o1[...] = z[:, 16:]


def _tc_z1(xp, W1p, dis2):
    return pl.pallas_call(
        _z1k,
        grid=(_NB_,),
        in_specs=[
            pl.BlockSpec((_BN_, 16), lambda i: (i, 0)),
            pl.BlockSpec((16, 32), lambda i: (0, 0)),
            pl.BlockSpec((_BN_, 1), lambda i: (i, 0)),
        ],
        out_specs=[pl.BlockSpec((_BN_, 16), lambda i: (i, 0))] * 2,
        out_shape=[jax.ShapeDtypeStruct((NR, 16), jnp.float32)] * 2,
    )(xp, W1p, dis2)


def _make_stats(P):
    def body(*refs):
        srefs = refs[:P]
        zrefs = refs[P:2 * P]
        dis_ref = refs[2 * P]
        b_ref = refs[2 * P + 1]
        hrefs = refs[2 * P + 2:3 * P + 2]
        st_ref = refs[3 * P + 2]
        i = pl.program_id(0)

        @pl.when(i == 0)
        def _():
            st_ref[...] = jnp.zeros_like(st_ref)

        rows = lax.broadcasted_iota(jnp.int32, (_BN_, 1), 0) + i * _BN_
        m = rows < N
        dis = dis_ref[...]
        for q in range(P):
            h = dis * (srefs[q][...] + zrefs[q][...]) + b_ref[0, 16 * q:16 * (q + 1)][None, :]
            hrefs[q][...] = h
            hm = jnp.where(m, h, 0.0)
            st_ref[0, 16 * q:16 * (q + 1)] += jnp.sum(hm, axis=0)
            st_ref[1, 16 * q:16 * (q + 1)] += jnp.sum(hm * hm, axis=0)

    def call(splanes, zplanes, dis2, brow):
        return pl.pallas_call(
            body,
            grid=(_NB_,),
            in_specs=(
                [pl.BlockSpec((_BN_, 16), lambda i: (i, 0))] * (2 * P)
                + [pl.BlockSpec((_BN_, 1), lambda i: (i, 0)),
                   pl.BlockSpec((1, 16 * P), lambda i: (0, 0))]
            ),
            out_specs=(
                [pl.BlockSpec((_BN_, 16), lambda i: (i, 0))] * P
                + [pl.BlockSpec((2, 16 * P), lambda i: (0, 0))]
            ),
            out_shape=(
                [jax.ShapeDtypeStruct((NR, 16), jnp.float32)] * P
                + [jax.ShapeDtypeStruct((2, 16 * P), jnp.float32)]
            ),
        )(*splanes, *zplanes, dis2, brow)

    return call


_stats2 = _make_stats(2)
_stats4 = _make_stats(4)


def _make_apply(P, PN):
    F, FN = 16 * P, 16 * PN

    def body(*refs):
        hrefs = refs[:P]
        st_ref, g_ref, be_ref, w_ref, dis_ref = refs[P:P + 5]
        orefs = refs[P + 5:]
        st = st_ref[...]
        full = None
        for q in range(P):
            mean = st[0, 16 * q:16 * (q + 1)] * (1.0 / N)
            var = st[1, 16 * q:16 * (q + 1)] * (1.0 / N) - mean * mean
            inv = lax.rsqrt(var + EPS)
            g = g_ref[0, 16 * q:16 * (q + 1)]
            be = be_ref[0, 16 * q:16 * (q + 1)]
            hn = (hrefs[q][...] - mean[None, :]) * (inv * g)[None, :] + be[None, :]
            hn = jnp.maximum(hn, 0.0)
            part = jnp.dot(hn, w_ref[16 * q:16 * (q + 1), :],
                           preferred_element_type=jnp.float32)
            full = part if full is None else full + part
        full = full * dis_ref[...]
        for qn in range(PN):
            orefs[qn][...] = full[:, 16 * qn:16 * (qn + 1)]

    def call(hplanes, st, grow, berow, W, dis2):
        return pl.pallas_call(
            body,
            grid=(_NB_,),
            in_specs=(
                [pl.BlockSpec((_BN_, 16), lambda i: (i, 0))] * P
                + [pl.BlockSpec((2, F), lambda i: (0, 0)),
                   pl.BlockSpec((1, F), lambda i: (0, 0)),
                   pl.BlockSpec((1, F), lambda i: (0, 0)),
                   pl.BlockSpec((F, FN), lambda i: (0, 0)),
                   pl.BlockSpec((_BN_, 1), lambda i: (i, 0))]
            ),
            out_specs=[pl.BlockSpec((_BN_, 16), lambda i: (i, 0))] * PN,
            out_shape=[jax.ShapeDtypeStruct((NR, 16), jnp.float32)] * PN,
        )(*hplanes, st, grow, berow, W, dis2)

    return call


_apply_2_4 = _make_apply(2, 4)
_apply_4_2 = _make_apply(4, 2)


def _final_body(h0, h1, st_ref, g_ref, be_ref, wo_ref, bo_ref, out_ref, mx):
    i = pl.program_id(0)

    @pl.when(i == 0)
    def _():
        mx[...] = jnp.full_like(mx, -1e30)

    rows = lax.broadcasted_iota(jnp.int32, (_BN_, 1), 0) + i * _BN_
    m = rows < N
    st = st_ref[...]
    for q, h_ref in enumerate((h0, h1)):
        mean = st[0, 16 * q:16 * (q + 1)] * (1.0 / N)
        var = st[1, 16 * q:16 * (q + 1)] * (1.0 / N) - mean * mean
        inv = lax.rsqrt(var + EPS)
        g = g_ref[0, 16 * q:16 * (q + 1)]
        be = be_ref[0, 16 * q:16 * (q + 1)]
        hn = (h_ref[...] - mean[None, :]) * (inv * g)[None, :] + be[None, :]
        hn = jnp.maximum(hn, 0.0)
        hn = jnp.where(m, hn, -1e30)
        cm = jnp.max(hn, axis=0)
        mx[0, 16 * q:16 * (q + 1)] = jnp.maximum(
            mx[0, 16 * q:16 * (q + 1)], cm)

    @pl.when(i == _NB_ - 1)
    def _():
        out_ref[...] = jnp.sum(mx[...] * wo_ref[...]).reshape(1, 1) + bo_ref[...]


def _tc_final(h3planes, st, grow, berow, worow, bo):
    return pl.pallas_call(
        _final_body,
        grid=(_NB_,),
        in_specs=(
            [pl.BlockSpec((_BN_, 16), lambda i: (i, 0))] * 2
            + [pl.BlockSpec((2, 32), lambda i: (0, 0)),
               pl.BlockSpec((1, 32), lambda i: (0, 0)),
               pl.BlockSpec((1, 32), lambda i: (0, 0)),
               pl.BlockSpec((1, 32), lambda i: (0, 0)),
               pl.BlockSpec((1, 1), lambda i: (0, 0))]
        ),
        out_specs=pl.BlockSpec((1, 1), lambda i: (0, 0)),
        out_shape=jax.ShapeDtypeStruct((1, 1), jnp.float32),
        scratch_shapes=[pltpu.VMEM((1, 32), jnp.float32)],
    )(*h3planes, st, grow, berow, worow, bo)


def kernel(x, edge_index, W1, b1, g1, be1, W2, b2, g2, be2, W3, b3, g3, be3, Wo, bo):
    src = edge_index[0].astype(jnp.int32)
    dst = edge_index[1].astype(jnp.int32)
    padi = jnp.full((E_PAD - E,), N, jnp.int32)
    src = jnp.concatenate([src, padi])
    dst = jnp.concatenate([dst, padi])

    zeros1 = jnp.zeros((NR,), jnp.float32)
    zeros2 = jnp.zeros((STRIPE, 16), jnp.float32)

    xp = jnp.pad(x, ((0, NR - N), (0, 16 - x.shape[1])))
    W1p = jnp.pad(W1, ((0, 16 - W1.shape[0]), (0, 0)))

    dst2 = dst.reshape(E_PAD // 128, 128)

    hist32 = _deg_call(dst, zeros1)
    dis2 = _tc_dis(hist32)

    z1 = _tc_z1(xp, W1p, dis2)
    s1 = _spmm2(src, dst2, zeros2, *z1)
    *h1, st1 = _stats2(s1, z1, dis2, b1.reshape(1, 32))

    z2 = _apply_2_4(h1, st1, g1.reshape(1, 32), be1.reshape(1, 32), W2, dis2)
    s2 = _spmm4(src, dst2, zeros2, *z2)
    *h2, st2 = _stats4(s2, z2, dis2, b2.reshape(1, 64))

    z3 = _apply_4_2(h2, st2, g2.reshape(1, 64), be2.reshape(1, 64), W3, dis2)
    s3 = _spmm2(src, dst2, zeros2, *z3)
    *h3, st3 = _stats2(s3, z3, dis2, b3.reshape(1, 32))

    out = _tc_final(h3, st3, g3.reshape(1, 32), be3.reshape(1, 32),
                    Wo.reshape(1, 32), bo.reshape(1, 1))
    return out[:, 0]


# speedup vs baseline: 1.3035x; 1.3035x over previous
import functools

import jax
import jax.numpy as jnp
from jax import lax
from jax.experimental import pallas as pl
from jax.experimental.pallas import tpu as pltpu
from jax.experimental.pallas import tpu_sc as plsc

N = 100000
NR = 100352
STRIPE = NR // 16
E = 6400000
ET = 401280
E_PAD = ET * 16
DEG_CH = 3344
DEG_ET = E_PAD // 32
DEG_NITER = DEG_ET // DEG_CH
EPS = 1e-5

_MESH = plsc.VectorSubcoreMesh(core_axis_name="c", subcore_axis_name="s")


def _deg_body(dst_hbm, zeros1, hist_out, hist, db0, db1, sem0, sem1):
    c = lax.axis_index("c")
    s = lax.axis_index("s")
    wid = c * 16 + s
    dbufs = (db0, db1)
    sems = (sem0, sem1)
    pltpu.sync_copy(zeros1, hist)
    ones = jnp.full((16,), 1.0, jnp.float32)

    def fire(t, b):
        pltpu.async_copy(
            dst_hbm.at[pl.ds(wid * DEG_ET + t * DEG_CH, DEG_CH)],
            dbufs[b], sems[b])

    def drain(b):
        pltpu.make_async_copy(
            dst_hbm.at[pl.ds(0, DEG_CH)], dbufs[b], sems[b]).wait()

    fire(0, 0)

    @pl.loop(0, DEG_NITER // 2)
    def _(u):
        for r in range(2):
            t = 2 * u + r

            @pl.when(t + 1 < DEG_NITER)
            def _():
                fire(t + 1, 1 - r)

            drain(r)
            for j in range(DEG_CH // 16):
                idx = dbufs[r][pl.ds(j * 16, 16)]
                plsc.addupdate_scatter(hist, [idx], ones)

    pltpu.sync_copy(hist, hist_out.at[pl.ds(wid * NR, NR)])


_deg_call = pl.kernel(
    _deg_body,
    out_type=jax.ShapeDtypeStruct((32 * NR,), jnp.float32),
    mesh=_MESH,
    compiler_params=pltpu.CompilerParams(needs_layout_passes=False),
    scratch_types=[
        pltpu.VMEM((NR,), jnp.float32),
        pltpu.VMEM((DEG_CH,), jnp.int32),
        pltpu.VMEM((DEG_CH,), jnp.int32),
        pltpu.SemaphoreType.DMA,
        pltpu.SemaphoreType.DMA,
    ],
)


GRP = 384
GK = GRP // 128
NGRP = ET // GRP
IB = 4
MAIN = (NGRP - 8) // IB


def _make_spmm(P):
    PH = P // 2

    def body(src_hbm, dst2_hbm, zeros2, *rest):
        zrefs = rest[:P]
        srefs = rest[P:2 * P]
        acc = rest[2 * P]
        pos = 2 * P + 1
        src_ch = rest[pos:pos + IB]
        pos += IB
        dst_ch = rest[pos:pos + IB]
        pos += IB
        row_ch = [rest[pos + b * GK:pos + (b + 1) * GK] for b in range(IB)]
        pos += IB * GK
        i_sem = rest[pos:pos + IB]
        g_sem = rest[pos + IB:pos + 2 * IB]
        s_sem = rest[pos + 2 * IB:pos + 3 * IB]
        c = lax.axis_index("c")
        s = lax.axis_index("s")

        def fire_i(u, b):
            base = s * ET + u * GRP
            pltpu.async_copy(src_hbm.at[pl.ds(base, GRP)], src_ch[b], i_sem[b])
            pltpu.async_copy(dst2_hbm.at[pl.ds(base // 128, GK)], dst_ch[b],
                             i_sem[b])

        def wait_i(b):
            pltpu.make_async_copy(
                src_hbm.at[pl.ds(0, GRP)], src_ch[b], i_sem[b]).wait()
            pltpu.make_async_copy(
                dst2_hbm.at[pl.ds(0, GK)], dst_ch[b], i_sem[b]).wait()

        def fire_g(b, Z):
            for m in range(GK):
                pltpu.async_copy(
                    Z.at[src_ch[b].at[pl.ds(128 * m, 128)]], row_ch[b][m],
                    g_sem[b])

        def wait_g(b, Z):
            for m in range(GK):
                pltpu.make_async_copy(
                    Z.at[src_ch[b].at[pl.ds(128 * m, 128)]], row_ch[b][m],
                    g_sem[b]).wait()

        def fire_s(b):
            for m in range(GK):
                pltpu.async_copy(row_ch[b][m], acc.at[dst_ch[b].at[m]],
                                 s_sem[b], add=True)

        def wait_s(b):
            for m in range(GK):
                pltpu.make_async_copy(row_ch[b][m], acc.at[dst_ch[b].at[m]],
                                      s_sem[b]).wait()

        for cv in range(2):
            @pl.when(c == cv)
            def _():
                for k in range(PH):
                    q = cv * PH + k
                    Z, S = zrefs[q], srefs[q]
                    pltpu.sync_copy(zeros2, acc.at[pl.ds(s * STRIPE, STRIPE)])
                    plsc.subcore_barrier()

                    def group(u, ib, has_gm1, has_sm2, has_ip2):
                        wait_i(ib)
                        fire_g(ib, Z)
                        if has_gm1:
                            wait_g((ib - 1) % IB, Z)
                            fire_s((ib - 1) % IB)
                        if has_sm2:
                            wait_s((ib - 2) % IB)
                        if has_ip2:
                            fire_i(u + 2, (ib + 2) % IB)

                    fire_i(0, 0)
                    fire_i(1, 1)
                    group(0, 0, False, False, True)
                    group(1, 1, True, False, True)

                    @pl.loop(0, MAIN)
                    def _(v):
                        for r in range(IB):
                            group(2 + IB * v + r, (2 + r) % IB,
                                  True, True, True)

                    for u in range(2 + IB * MAIN, NGRP):
                        group(u, u % IB, True, True, u + 2 < NGRP)
                    wait_g((NGRP - 1) % IB, Z)
                    fire_s((NGRP - 1) % IB)
                    wait_s((NGRP - 2) % IB)
                    wait_s((NGRP - 1) % IB)
                    plsc.subcore_barrier()
                    pltpu.sync_copy(acc.at[pl.ds(s * STRIPE, STRIPE)],
                                    S.at[pl.ds(s * STRIPE, STRIPE)])
                    plsc.subcore_barrier()

    return pl.kernel(
        body,
        out_type=[jax.ShapeDtypeStruct((NR, 16), jnp.float32)] * P,
        mesh=_MESH,
        compiler_params=pltpu.CompilerParams(
            needs_layout_passes=False, use_tc_tiling_on_sc=False),
        scratch_types=(
            [pltpu.VMEM_SHARED((NR, 16), jnp.float32)]
            + [pltpu.VMEM((GRP,), jnp.int32) for _ in range(IB)]
            + [pltpu.VMEM((GK, 128), jnp.int32) for _ in range(IB)]
            + [pltpu.VMEM((128, 16), jnp.float32) for _ in range(IB * GK)]
            + [pltpu.SemaphoreType.DMA for _ in range(3 * IB)]
        ),
    )


_spmm2 = _make_spmm(2)
_spmm4 = _make_spmm(4)


NR8 = NR // 8
_BN8 = 448
_NB8 = NR8 // _BN8
_PADROW = N // 8


def _disk(hist_ref, dis_ref):
    deg = jnp.sum(hist_ref[...], axis=0) + 1.0
    dis_ref[...] = lax.rsqrt(deg)


def _tc_dis(hist32):
    h = hist32.reshape(32, NR // 128, 128)
    dis = pl.pallas_call(
        _disk,
        out_shape=jax.ShapeDtypeStruct((NR // 128, 128), jnp.float32),
    )(h)
    return jnp.repeat(dis.reshape(-1), 16).reshape(NR8, 128)


def _z1k(x_ref, w_ref, dis_ref, o0, o1):
    x = x_ref[...]
    d = dis_ref[...]
    for qn, o in enumerate((o0, o1)):
        o[...] = jnp.dot(x, w_ref[qn], preferred_element_type=jnp.float32) * d


def _tc_z1(x128, W1big, disE):
    return pl.pallas_call(
        _z1k,
        grid=(_NB8,),
        in_specs=[
            pl.BlockSpec((_BN8, 128), lambda i: (i, 0)),
            pl.BlockSpec((2, 128, 128), lambda i: (0, 0, 0)),
            pl.BlockSpec((_BN8, 128), lambda i: (i, 0)),
        ],
        out_specs=[pl.BlockSpec((_BN8, 128), lambda i: (i, 0))] * 2,
        out_shape=[jax.ShapeDtypeStruct((NR8, 128), jnp.float32)] * 2,
    )(x128, W1big, disE)


def _make_stats(P):
    def body(*refs):
        srefs = refs[:P]
        zrefs = refs[P:2 * P]
        dis_ref = refs[2 * P]
        b_ref = refs[2 * P + 1]
        hrefs = refs[2 * P + 2:3 * P + 2]
        st_ref = refs[3 * P + 2]
        i = pl.program_id(0)

        @pl.when(i == 0)
        def _():
            st_ref[...] = jnp.zeros_like(st_ref)

        rows = lax.broadcasted_iota(jnp.int32, (_BN8, 1), 0) + i * _BN8
        m = rows < _PADROW
        dis = dis_ref[...]
        for q in range(P):
            h = dis * (srefs[q][...] + zrefs[q][...]) + b_ref[q][None, :]
            hrefs[q][...] = h
            hm = jnp.where(m, h, 0.0)
            st_ref[0, 128 * q:128 * (q + 1)] += jnp.sum(hm, axis=0)
            st_ref[1, 128 * q:128 * (q + 1)] += jnp.sum(hm * hm, axis=0)

    def call(splanes, zplanes, disE, b128):
        return pl.pallas_call(
            body,
            grid=(_NB8,),
            in_specs=(
                [pl.BlockSpec((_BN8, 128), lambda i: (i, 0))] * (2 * P)
                + [pl.BlockSpec((_BN8, 128), lambda i: (i, 0)),
                   pl.BlockSpec((P, 128), lambda i: (0, 0))]
            ),
            out_specs=(
                [pl.BlockSpec((_BN8, 128), lambda i: (i, 0))] * P
                + [pl.BlockSpec((2, 128 * P), lambda i: (0, 0))]
            ),
            out_shape=(
                [jax.ShapeDtypeStruct((NR8, 128), jnp.float32)] * P
                + [jax.ShapeDtypeStruct((2, 128 * P), jnp.float32)]
            ),
        )(*splanes, *zplanes, disE, b128)

    return call


_stats2 = _make_stats(2)
_stats4 = _make_stats(4)


def _bn_consts(st, q):
    tot = st[0, 128 * q:128 * q + 16]
    tsq = st[1, 128 * q:128 * q + 16]
    for a in range(1, 8):
        tot = tot + st[0, 128 * q + 16 * a:128 * q + 16 * a + 16]
        tsq = tsq + st[1, 128 * q + 16 * a:128 * q + 16 * a + 16]
    mean = tot * (1.0 / N)
    var = tsq * (1.0 / N) - mean * mean
    inv = lax.rsqrt(var + EPS)
    mean128 = jnp.concatenate([mean] * 8)
    inv128 = jnp.concatenate([inv] * 8)
    return mean128, inv128


def _make_apply(P, PN):
    def body(*refs):
        hrefs = refs[:P]
        st_ref, g_ref, be_ref, w_ref, dis_ref = refs[P:P + 5]
        orefs = refs[P + 5:]
        st = st_ref[...]
        hn = []
        for q in range(P):
            mean128, inv128 = _bn_consts(st, q)
            z = (hrefs[q][...] - mean128[None, :]) * (inv128 * g_ref[q])[None, :]
            hn.append(jnp.maximum(z + be_ref[q][None, :], 0.0))
        d = dis_ref[...]
        for qn in range(PN):
            acc = jnp.dot(hn[0], w_ref[0, qn],
                          preferred_element_type=jnp.float32)
            for q in range(1, P):
                acc = acc + jnp.dot(hn[q], w_ref[q, qn],
                                    preferred_element_type=jnp.float32)
            orefs[qn][...] = acc * d

    def call(hplanes, st, g128, be128, Wbig, disE):
        return pl.pallas_call(
            body,
            grid=(_NB8,),
            in_specs=(
                [pl.BlockSpec((_BN8, 128), lambda i: (i, 0))] * P
                + [pl.BlockSpec((2, 128 * P), lambda i: (0, 0)),
                   pl.BlockSpec((P, 128), lambda i: (0, 0)),
                   pl.BlockSpec((P, 128), lambda i: (0, 0)),
                   pl.BlockSpec((P, PN, 128, 128), lambda i: (0, 0, 0, 0)),
                   pl.BlockSpec((_BN8, 128), lambda i: (i, 0))]
            ),
            out_specs=[pl.BlockSpec((_BN8, 128), lambda i: (i, 0))] * PN,
            out_shape=[jax.ShapeDtypeStruct((NR8, 128), jnp.float32)] * PN,
        )(*hplanes, st, g128, be128, Wbig, disE)

    return call


_apply_2_4 = _make_apply(2, 4)
_apply_4_2 = _make_apply(4, 2)


def _final_body(h0, h1, st_ref, g_ref, be_ref, wo_ref, bo_ref, out_ref, mx):
    i = pl.program_id(0)

    @pl.when(i == 0)
    def _():
        mx[...] = jnp.full_like(mx, -1e30)

    rows = lax.broadcasted_iota(jnp.int32, (_BN8, 1), 0) + i * _BN8
    m = rows < _PADROW
    st = st_ref[...]
    for q, h_ref in enumerate((h0, h1)):
        mean128, inv128 = _bn_consts(st, q)
        hn = (h_ref[...] - mean128[None, :]) * (inv128 * g_ref[q])[None, :]
        hn = jnp.maximum(hn + be_ref[q][None, :], 0.0)
        hn = jnp.where(m, hn, -1e30)
        cm = jnp.max(hn, axis=0)
        mx[0, 128 * q:128 * (q + 1)] = jnp.maximum(
            mx[0, 128 * q:128 * (q + 1)], cm)

    @pl.when(i == _NB8 - 1)
    def _():
        tot = jnp.zeros((), jnp.float32)
        for q in range(2):
            gm = mx[0, 128 * q:128 * q + 16]
            for a in range(1, 8):
                gm = jnp.maximum(gm, mx[0, 128 * q + 16 * a:128 * q + 16 * a + 16])
            tot = tot + jnp.sum(gm * wo_ref[0, 16 * q:16 * (q + 1)])
        out_ref[...] = tot.reshape(1, 1) + bo_ref[...]


def _tc_final(h3planes, st, g128, be128, worow, bo):
    return pl.pallas_call(
        _final_body,
        grid=(_NB8,),
        in_specs=(
            [pl.BlockSpec((_BN8, 128), lambda i: (i, 0))] * 2
            + [pl.BlockSpec((2, 256), lambda i: (0, 0)),
               pl.BlockSpec((2, 128), lambda i: (0, 0)),
               pl.BlockSpec((2, 128), lambda i: (0, 0)),
               pl.BlockSpec((1, 32), lambda i: (0, 0)),
               pl.BlockSpec((1, 1), lambda i: (0, 0))]
        ),
        out_specs=pl.BlockSpec((1, 1), lambda i: (0, 0)),
        out_shape=jax.ShapeDtypeStruct((1, 1), jnp.float32),
        scratch_shapes=[pltpu.VMEM((1, 256), jnp.float32)],
    )(*h3planes, st, g128, be128, worow, bo)


_EYE8 = None


def _bigW(W, P, PN):
    eye8 = jnp.eye(8, dtype=jnp.float32)
    t = W.reshape(P, 16, PN, 16).transpose(0, 2, 1, 3)
    big = (eye8[None, None, :, None, :, None]
           * t[:, :, None, :, None, :])
    return big.reshape(P, PN, 128, 128)


def _tile8(v, P):
    return jnp.tile(v.reshape(P, 16), (1, 8))


def kernel(x, edge_index, W1, b1, g1, be1, W2, b2, g2, be2, W3, b3, g3, be3, Wo, bo):
    src = edge_index[0].astype(jnp.int32)
    dst = edge_index[1].astype(jnp.int32)
    padi = jnp.full((E_PAD - E,), N, jnp.int32)
    src = jnp.concatenate([src, padi])
    dst = jnp.concatenate([dst, padi])

    zeros1 = jnp.zeros((NR,), jnp.float32)
    zeros2 = jnp.zeros((STRIPE, 16), jnp.float32)

    xp = jnp.pad(x, ((0, NR - N), (0, 16 - x.shape[1])))
    x128 = xp.reshape(NR8, 128)
    W1p = jnp.pad(W1, ((0, 16 - W1.shape[0]), (0, 0)))
    W1big = _bigW(W1p, 1, 2)[0]
    W2big = _bigW(W2, 2, 4)
    W3big = _bigW(W3, 4, 2)

    dst2 = dst.reshape(E_PAD // 128, 128)

    hist32 = _deg_call(dst, zeros1)
    disE = _tc_dis(hist32)

    z1 = _tc_z1(x128, W1big, disE)
    s1 = _spmm2(src, dst2, zeros2, *[z.reshape(NR, 16) for z in z1])
    *h1, st1 = _stats2([s.reshape(NR8, 128) for s in s1], z1, disE,
                       _tile8(b1, 2))

    z2 = _apply_2_4(h1, st1, _tile8(g1, 2), _tile8(be1, 2), W2big, disE)
    s2 = _spmm4(src, dst2, zeros2, *[z.reshape(NR, 16) for z in z2])
    *h2, st2 = _stats4([s.reshape(NR8, 128) for s in s2], z2, disE,
                       _tile8(b2, 4))

    z3 = _apply_4_2(h2, st2, _tile8(g2, 4), _tile8(be2, 4), W3big, disE)
    s3 = _spmm2(src, dst2, zeros2, *[z.reshape(NR, 16) for z in z3])
    *h3, st3 = _stats2([s.reshape(NR8, 128) for s in s3], z3, disE,
                       _tile8(b3, 2))

    out = _tc_final(h3, st3, _tile8(g3, 2), _tile8(be3, 2),
                    Wo.reshape(1, 32), bo.reshape(1, 1))
    return out[:, 0]
